# baseline (speedup 1.0000x reference)
import functools

import jax
import jax.numpy as jnp
from jax import lax
from jax.experimental import pallas as pl
from jax.experimental.pallas import tpu as pltpu
from jax.experimental.pallas import tpu_sc as plsc

N = 65536
L = 10
V = 10
NC = 2
NS = 16
NW = NC * NS
N_PER_W = N // NW
CHUNK_N = 256
CHUNKS = N_PER_W // CHUNK_N
GROUPS = CHUNK_N // 16
NBUF = 3


def _ci(v):
    return lax.full((16,), v, jnp.int32)


def _cf(v):
    return lax.full((16,), v, jnp.float32)


@functools.partial(
    pl.kernel,
    out_type=jax.ShapeDtypeStruct((L, V, N), jnp.float32),
    mesh=plsc.VectorSubcoreMesh(core_axis_name="c", subcore_axis_name="s"),
    scratch_types=(
        [pltpu.VMEM((L, V, CHUNK_N), jnp.float32)] * NBUF
        + [pltpu.SemaphoreType.DMA] * (2 * NBUF + 1)
    ),
    compiler_params=pltpu.CompilerParams(needs_layout_passes=False,
                                         use_tc_tiling_on_sc=True),
)
def _solve(x_hbm, out_hbm, *scratch):
    bufs = scratch[:NBUF]
    sin = scratch[NBUF:2 * NBUF]
    sout = scratch[2 * NBUF:3 * NBUF]
    l0sem = scratch[3 * NBUF]
    wid = lax.axis_index("s") * NC + lax.axis_index("c")
    base_n = wid * N_PER_W
    lanes = lax.iota(jnp.int32, 16)

    def src(ci):
        return x_hbm.at[:, :, pl.ds(base_n + ci * CHUNK_N, CHUNK_N)]

    def dst(ci):
        return out_hbm.at[:, :, pl.ds(base_n + ci * CHUNK_N, CHUNK_N)]

    def compute(buf):
        def group_body(gi, c2):
            g0 = gi * 16
            orall = _ci(0)
            for l in range(L):
                m = buf[l, 0, pl.ds(g0, 16)]
                b = _ci(1)
                for v in range(1, V):
                    xv = buf[l, v, pl.ds(g0, 16)]
                    gt = xv > m
                    b = jnp.where(gt, _ci(1 << v), b)
                    m = jnp.where(gt, xv, m)
                orall = orall | b
            nidx = lanes + g0
            one = _cf(1.0)
            zero = _cf(0.0)
            for l in range(1, L):
                miss = (orall & _ci(1 << (l - 1))) == _ci(0)
                for v in range(V):
                    val = one if v == (l - 1) else zero
                    plsc.store_scatter(buf, [_ci(l), _ci(v), nidx], val,
                                       mask=miss)
            return c2

        lax.fori_loop(0, GROUPS, group_body, 0)

    AHEAD = NBUF - 1
    l0 = pltpu.async_copy(
        x_hbm.at[0, :, pl.ds(base_n, N_PER_W)],
        out_hbm.at[0, :, pl.ds(base_n, N_PER_W)], l0sem)
    for pi in range(AHEAD):
        pltpu.async_copy(src(pi), bufs[pi], sin[pi])
    for ci in range(CHUNKS):
        b = ci % NBUF
        if ci + AHEAD < CHUNKS:
            nb = (ci + AHEAD) % NBUF
            if ci >= 1:
                pltpu.make_async_copy(bufs[nb].at[pl.ds(1, L - 1)],
                                      dst(ci - 1).at[pl.ds(1, L - 1)],
                                      sout[nb]).wait()
            pltpu.async_copy(src(ci + AHEAD), bufs[nb], sin[nb])
        pltpu.make_async_copy(src(ci), bufs[b], sin[b]).wait()
        compute(bufs[b])
        pltpu.async_copy(bufs[b].at[pl.ds(1, L - 1)],
                         dst(ci).at[pl.ds(1, L - 1)], sout[b])
    for ci in range(max(0, CHUNKS - AHEAD), CHUNKS):
        pltpu.make_async_copy(bufs[ci % NBUF].at[pl.ds(1, L - 1)],
                              dst(ci).at[pl.ds(1, L - 1)],
                              sout[ci % NBUF]).wait()
    l0.wait()


def kernel(memory_vb):
    xt = memory_vb.transpose(1, 2, 0)
    out = _solve(xt)
    return out.transpose(2, 0, 1)


# speedup vs baseline: 2.7857x; 2.7857x over previous
import functools

import jax
import jax.numpy as jnp
from jax import lax
from jax.experimental import pallas as pl
from jax.experimental.pallas import tpu as pltpu
from jax.experimental.pallas import tpu_sc as plsc

N = 65536
L = 10
V = 10
NC = 2
NS = 16
NW = NC * NS
N_PER_W = N // NW
CHUNK_N = 256
CHUNKS = N_PER_W // CHUNK_N
GROUPS = CHUNK_N // 16
NBUF = 3


def _ci(v):
    return lax.full((16,), v, jnp.int32)


def _cf(v):
    return lax.full((16,), v, jnp.float32)


@functools.partial(
    pl.kernel,
    out_type=jax.ShapeDtypeStruct((L, V, N), jnp.float32),
    mesh=plsc.VectorSubcoreMesh(core_axis_name="c", subcore_axis_name="s"),
    scratch_types=(
        [pltpu.VMEM((L, V, CHUNK_N), jnp.float32)] * NBUF
        + [pltpu.SemaphoreType.DMA] * (2 * NBUF)
    ),
    compiler_params=pltpu.CompilerParams(needs_layout_passes=False,
                                         use_tc_tiling_on_sc=True),
)
def _solve(x_hbm, out_hbm, *scratch):
    bufs = scratch[:NBUF]
    sin = scratch[NBUF:2 * NBUF]
    sout = scratch[2 * NBUF:]
    wid = lax.axis_index("s") * NC + lax.axis_index("c")
    base_n = wid * N_PER_W
    lanes = lax.iota(jnp.int32, 16)

    def src(ci):
        return x_hbm.at[:, :, pl.ds(base_n + ci * CHUNK_N, CHUNK_N)]

    def dst(ci):
        return out_hbm.at[:, :, pl.ds(base_n + ci * CHUNK_N, CHUNK_N)]

    def compute(buf):
        def group_body(gi, c2):
            g0 = gi * 16
            orall = _ci(0)
            for l in range(L):
                m = buf[l, 0, pl.ds(g0, 16)]
                b = _ci(1)
                for v in range(1, V):
                    xv = buf[l, v, pl.ds(g0, 16)]
                    gt = xv > m
                    b = jnp.where(gt, _ci(1 << v), b)
                    m = jnp.where(gt, xv, m)
                orall = orall | b
            nidx = lanes + g0
            one = _cf(1.0)
            zero = _cf(0.0)
            for l in range(1, L):
                miss = (orall & _ci(1 << (l - 1))) == _ci(0)
                for v in range(V):
                    val = one if v == (l - 1) else zero
                    plsc.store_scatter(buf, [_ci(l), _ci(v), nidx], val,
                                       mask=miss)
            return c2

        lax.fori_loop(0, GROUPS, group_body, 0)

    AHEAD = NBUF - 1
    for pi in range(AHEAD):
        pltpu.async_copy(src(pi), bufs[pi], sin[pi])
    for ci in range(CHUNKS):
        b = ci % NBUF
        if ci + AHEAD < CHUNKS:
            nb = (ci + AHEAD) % NBUF
            if ci >= 1:
                pltpu.make_async_copy(bufs[nb], dst(ci - 1), sout[nb]).wait()
            pltpu.async_copy(src(ci + AHEAD), bufs[nb], sin[nb])
        pltpu.make_async_copy(src(ci), bufs[b], sin[b]).wait()
        compute(bufs[b])
        pltpu.async_copy(bufs[b], dst(ci), sout[b])
    for ci in range(max(0, CHUNKS - AHEAD), CHUNKS):
        pltpu.make_async_copy(bufs[ci % NBUF], dst(ci), sout[ci % NBUF]).wait()


def kernel(memory_vb):
    xt = memory_vb.transpose(1, 2, 0)
    out = _solve(xt)
    return out.transpose(2, 0, 1)
